# baseline (speedup 1.0000x reference)
import jax
import jax.numpy as jnp
from jax import lax
from jax.experimental import pallas as pl
from jax.experimental.pallas import tpu as pltpu
from jax.experimental.pallas import tpu_sc as plsc

L = 20
B = 1024
T = 1000
N = L * B
NC = 2
NS = 16
NW = NC * NS
ROWS_W = N // NW
CHUNK = 16
CWORDS = CHUNK * T
NBUF = 4
NCHUNK = ROWS_W // CHUNK
NROUND = NCHUNK // NBUF


def _body(x_hbm, out_hbm, x_v, b0_v, b1_v, b2_v, b3_v, s0, s1, s2, s3):
    bufs = (b0_v, b1_v, b2_v, b3_v)
    sems = (s0, s1, s2, s3)
    wid = lax.axis_index("s") * NC + lax.axis_index("c")

    pltpu.sync_copy(x_hbm, x_v)

    zeros16 = jnp.zeros((16,), jnp.float32)

    def zbody(i, c):
        for k in range(NBUF):
            bufs[k][pl.ds(i * 16, 16)] = zeros16
        return c

    lax.fori_loop(0, CWORDS // 16, zbody, 0, unroll=4)

    lane = lax.iota(jnp.int32, 16)
    ones16 = jnp.ones((16,), jnp.float32)
    rowbase = lane * T
    ctxbase = rowbase + (T - 1)
    row0 = wid * ROWS_W

    def fill(buf, n0):
        l = n0 // B
        bg = n0 - l * B
        xv = x_v[pl.ds(l * B + bg, 16)]
        valid = xv >= 0

        cnt = jnp.zeros((16,), jnp.int32)
        for j in range(L):
            xj = x_v[pl.ds(j * B + bg, 16)]
            hit = (xj == xv) & (j <= l)
            cnt = cnt + hit.astype(jnp.int32)
        yv = (valid & ((cnt & 1) == 0)).astype(jnp.float32)

        plsc.store_scatter(buf, [rowbase + xv], ones16, mask=valid)
        plsc.addupdate_scatter(buf, [ctxbase], yv)

    def undo(buf, n0):
        l = n0 // B
        bg = n0 - l * B
        xv = x_v[pl.ds(l * B + bg, 16)]
        plsc.store_scatter(buf, [ctxbase], zeros16)
        plsc.store_scatter(buf, [rowbase + xv], zeros16, mask=xv >= 0)

    for k in range(NBUF):
        n0 = row0 + k * CHUNK
        fill(bufs[k], n0)
        pltpu.async_copy(bufs[k], out_hbm.at[pl.ds(n0 * T, CWORDS)], sems[k])

    def round_body(r, c):
        for k in range(NBUF):
            n0 = row0 + (r * NBUF + k) * CHUNK
            prev = n0 - NBUF * CHUNK
            pltpu.make_async_copy(
                bufs[k], out_hbm.at[pl.ds(prev * T, CWORDS)], sems[k]
            ).wait()
            undo(bufs[k], prev)
            fill(bufs[k], n0)
            pltpu.async_copy(bufs[k], out_hbm.at[pl.ds(n0 * T, CWORDS)], sems[k])
        return c

    lax.fori_loop(1, NROUND, round_body, 0)

    for k in range(NBUF):
        n0 = row0 + ((NROUND - 1) * NBUF + k) * CHUNK
        pltpu.make_async_copy(
            bufs[k], out_hbm.at[pl.ds(n0 * T, CWORDS)], sems[k]
        ).wait()


_mesh = plsc.VectorSubcoreMesh(
    core_axis_name="c", subcore_axis_name="s", num_cores=NC, num_subcores=NS
)

_sc_call = pl.kernel(
    _body,
    out_type=jax.ShapeDtypeStruct((N * T,), jnp.float32),
    mesh=_mesh,
    scratch_types=[pltpu.VMEM((N,), jnp.int32)]
    + [pltpu.VMEM((CWORDS,), jnp.float32) for _ in range(NBUF)]
    + [pltpu.SemaphoreType.DMA for _ in range(NBUF)],
    compiler_params=pltpu.CompilerParams(needs_layout_passes=False),
)


@jax.jit
def kernel(x):
    x32 = x.astype(jnp.int32).reshape(-1)
    out = _sc_call(x32)
    return out.reshape(L, B, T)


# speedup vs baseline: 1.5728x; 1.5728x over previous
import jax
import jax.numpy as jnp
from jax import lax
from jax.experimental import pallas as pl
from jax.experimental.pallas import tpu as pltpu
from jax.experimental.pallas import tpu_sc as plsc

L = 20
B = 1024
T = 1000
N = L * B
NC = 2
NS = 16
NW = NC * NS
CHB = 64
BLK_PER_L = B // CHB
NBLK = L * BLK_PER_L
BLK_W = NBLK // NW


def _body(x_hbm, out_hbm, x_v, buf_v):
    wid = lax.axis_index("s") * NC + lax.axis_index("c")

    pltpu.sync_copy(x_hbm, x_v)

    zeros16 = jnp.zeros((16,), jnp.float32)

    def zbody(r, c):
        for i in range(T // 16):
            buf_v[r, pl.ds(i * 16, 16)] = zeros16
        buf_v[r, pl.ds(T - 16, 16)] = zeros16
        return c

    lax.fori_loop(0, CHB, zbody, 0)

    lane = lax.iota(jnp.int32, 16)
    ones16 = jnp.ones((16,), jnp.float32)
    ctxcol = jnp.full((16,), T - 1, jnp.int32)
    blk0 = wid * BLK_W

    def block_body(bi, c):
        n = blk0 + bi
        l = n // BLK_PER_L
        b0 = (n - l * BLK_PER_L) * CHB

        for g in range(CHB // 16):
            bg = b0 + g * 16
            xv = x_v[pl.ds(l * B + bg, 16)]
            valid = xv >= 0

            cnt = jnp.zeros((16,), jnp.int32)
            for j in range(L):
                xj = x_v[pl.ds(j * B + bg, 16)]
                hit = (xj == xv) & (j <= l)
                cnt = cnt + hit.astype(jnp.int32)
            yv = (valid & ((cnt & 1) == 0)).astype(jnp.float32)

            rows = g * 16 + lane
            plsc.store_scatter(buf_v, [rows, xv], ones16, mask=valid)
            plsc.addupdate_scatter(buf_v, [rows, ctxcol], yv)

        pltpu.sync_copy(buf_v, out_hbm.at[l, pl.ds(b0, CHB)])

        for g in range(CHB // 16):
            bg = b0 + g * 16
            xv = x_v[pl.ds(l * B + bg, 16)]
            rows = g * 16 + lane
            plsc.store_scatter(buf_v, [rows, ctxcol], zeros16)
            plsc.store_scatter(buf_v, [rows, xv], zeros16, mask=xv >= 0)
        return c

    lax.fori_loop(0, BLK_W, block_body, 0)


_mesh = plsc.VectorSubcoreMesh(
    core_axis_name="c", subcore_axis_name="s", num_cores=NC, num_subcores=NS
)

_sc_call = pl.kernel(
    _body,
    out_type=jax.ShapeDtypeStruct((L, B, T), jnp.float32),
    mesh=_mesh,
    scratch_types=[
        pltpu.VMEM((N,), jnp.int32),
        pltpu.VMEM((CHB, T), jnp.float32),
    ],
    compiler_params=pltpu.CompilerParams(needs_layout_passes=False),
)


@jax.jit
def kernel(x):
    x32 = x.astype(jnp.int32).reshape(-1)
    return _sc_call(x32)


# speedup vs baseline: 1.5918x; 1.0121x over previous
import jax
import jax.numpy as jnp
from jax import lax
from jax.experimental import pallas as pl
from jax.experimental.pallas import tpu as pltpu
from jax.experimental.pallas import tpu_sc as plsc

L = 20
B = 1024
T = 1000
N = L * B
NC = 2
NS = 16
NW = NC * NS
CHB = 64
BLK_PER_L = B // CHB
NBLK = L * BLK_PER_L
BLK_W = NBLK // NW


def _body(x_hbm, out_hbm, x_v, buf_v):
    wid = lax.axis_index("s") * NC + lax.axis_index("c")

    pltpu.sync_copy(x_hbm, x_v)

    zeros16 = jnp.zeros((16,), jnp.float32)

    def zbody(r, c):
        for i in range(T // 16):
            buf_v[r, pl.ds(i * 16, 16)] = zeros16
        buf_v[r, pl.ds(T - 16, 16)] = zeros16
        return c

    lax.fori_loop(0, CHB, zbody, 0)

    lane = lax.iota(jnp.int32, 16)
    ones16 = jnp.ones((16,), jnp.float32)
    ctxcol = jnp.full((16,), T - 1, jnp.int32)
    blk0 = wid * BLK_W

    def block_body(bi, c):
        n = blk0 + bi
        l = n // BLK_PER_L
        b0 = (n - l * BLK_PER_L) * CHB

        for g in range(CHB // 16):
            bg = b0 + g * 16
            xv = x_v[pl.ds(l * B + bg, 16)]
            valid = xv >= 0

            cnt = jnp.zeros((16,), jnp.int32)
            for j in range(L):
                xj = x_v[pl.ds(j * B + bg, 16)]
                hit = (xj == xv) & (j <= l)
                cnt = cnt + hit.astype(jnp.int32)
            yv = (valid & ((cnt & 1) == 0)).astype(jnp.float32)

            rows = g * 16 + lane
            plsc.store_scatter(buf_v, [rows, xv], ones16, mask=valid)
            plsc.addupdate_scatter(buf_v, [rows, ctxcol], yv)

        pltpu.sync_copy(buf_v, out_hbm.at[l, pl.ds(b0, CHB)])

        for g in range(CHB // 16):
            bg = b0 + g * 16
            xv = x_v[pl.ds(l * B + bg, 16)]
            rows = g * 16 + lane
            plsc.store_scatter(buf_v, [rows, ctxcol], zeros16)
            plsc.store_scatter(buf_v, [rows, xv], zeros16, mask=xv >= 0)
        return c

    lax.fori_loop(0, BLK_W, block_body, 0)


_mesh = plsc.VectorSubcoreMesh(
    core_axis_name="c", subcore_axis_name="s", num_cores=NC, num_subcores=NS
)

_sc_call = pl.kernel(
    _body,
    out_type=jax.ShapeDtypeStruct((L, B, T), jnp.float32),
    mesh=_mesh,
    scratch_types=[
        pltpu.VMEM((N,), jnp.int32),
        pltpu.VMEM((CHB, T), jnp.float32),
    ],
    compiler_params=pltpu.CompilerParams(
        needs_layout_passes=False, use_tc_tiling_on_sc=True
    ),
)


@jax.jit
def kernel(x):
    x32 = x.astype(jnp.int32).reshape(-1)
    return _sc_call(x32)


# speedup vs baseline: 3.2538x; 2.0442x over previous
import jax
import jax.numpy as jnp
from jax import lax
from jax.experimental import pallas as pl
from jax.experimental.pallas import tpu as pltpu
from jax.experimental.pallas import tpu_sc as plsc

L = 20
B = 1024
T = 1000
N = L * B
NC = 2
NS = 16
NW = NC * NS
TCR = 40
NT = T // TCR
NBLK = L * NT
NG = B // 16


def _body(x_hbm, out_hbm, x_v, b0_v, b1_v, s0, s1):
    bufs = (b0_v, b1_v)
    sems = (s0, s1)
    wid = lax.axis_index("s") * NC + lax.axis_index("c")

    pltpu.sync_copy(x_hbm, x_v)

    zeros16 = jnp.zeros((16,), jnp.float32)

    def zbody(q, c):
        r = q // NG
        g = q - r * NG
        for k in range(2):
            bufs[k][r, pl.ds(g * 16, 16)] = zeros16
        return c

    lax.fori_loop(0, TCR * NG, zbody, 0)

    lane = lax.iota(jnp.int32, 16)
    ones16 = jnp.ones((16,), jnp.float32)

    start = (wid * NBLK) // NW
    n = ((wid + 1) * NBLK) // NW - start

    def loc(i):
        m = start + i
        l = m // NT
        t0 = (m - l * NT) * TCR
        return l, t0

    def fill(buf, i):
        l, t0 = loc(i)
        xbase = l * B

        def gbody(g, c):
            xv = x_v[pl.ds(xbase + g * 16, 16)]
            rel = xv - t0
            inb = (rel >= 0) & (rel < TCR)
            plsc.store_scatter(buf, [rel, g * 16 + lane], ones16, mask=inb)
            return c

        lax.fori_loop(0, NG, gbody, 0)

        @pl.when(t0 == T - TCR)
        def _():
            def cbody(g, c):
                xv = x_v[pl.ds(xbase + g * 16, 16)]
                valid = xv >= 0
                cnt = jnp.zeros((16,), jnp.int32)
                for j in range(L):
                    xj = x_v[pl.ds(j * B + g * 16, 16)]
                    hit = (xj == xv) & (j <= l)
                    cnt = cnt + hit.astype(jnp.int32)
                yv = (valid & ((cnt & 1) == 0)).astype(jnp.float32)
                cs = pl.ds(g * 16, 16)
                buf[TCR - 1, cs] = buf[TCR - 1, cs] + yv
                return c

            lax.fori_loop(0, NG, cbody, 0)

    def undo(buf, i):
        l, t0 = loc(i)
        xbase = l * B

        def gbody(g, c):
            xv = x_v[pl.ds(xbase + g * 16, 16)]
            rel = xv - t0
            inb = (rel >= 0) & (rel < TCR)
            plsc.store_scatter(buf, [rel, g * 16 + lane], zeros16, mask=inb)
            return c

        lax.fori_loop(0, NG, gbody, 0)

        @pl.when(t0 == T - TCR)
        def _():
            def zctx(g, c):
                buf[TCR - 1, pl.ds(g * 16, 16)] = zeros16
                return c

            lax.fori_loop(0, NG, zctx, 0)

    def start_dma(buf, sem, i):
        l, t0 = loc(i)
        pltpu.async_copy(buf, out_hbm.at[l, pl.ds(t0, TCR)], sem)

    def wait_dma(buf, sem, i):
        l, t0 = loc(i)
        pltpu.make_async_copy(buf, out_hbm.at[l, pl.ds(t0, TCR)], sem).wait()

    for k in range(2):
        fill(bufs[k], k)
        start_dma(bufs[k], sems[k], k)

    def round_body(r, c):
        for k in range(2):
            i = 2 + 2 * (r - 1) + k
            wait_dma(bufs[k], sems[k], i - 2)
            undo(bufs[k], i - 2)
            fill(bufs[k], i)
            start_dma(bufs[k], sems[k], i)
        return c

    lax.fori_loop(1, (n - 2) // 2 + 1, round_body, 0)

    @pl.when((n & 1) == 1)
    def _():
        wait_dma(bufs[0], sems[0], n - 3)
        undo(bufs[0], n - 3)
        fill(bufs[0], n - 1)
        start_dma(bufs[0], sems[0], n - 1)

    wait_dma(bufs[0], sems[0], n - 1 - ((n - 1) & 1))
    wait_dma(bufs[1], sems[1], n - 1 - (n & 1))


_mesh = plsc.VectorSubcoreMesh(
    core_axis_name="c", subcore_axis_name="s", num_cores=NC, num_subcores=NS
)

_sc_call = pl.kernel(
    _body,
    out_type=jax.ShapeDtypeStruct((L, T, B), jnp.float32),
    mesh=_mesh,
    scratch_types=[
        pltpu.VMEM((N,), jnp.int32),
        pltpu.VMEM((TCR, B), jnp.float32),
        pltpu.VMEM((TCR, B), jnp.float32),
        pltpu.SemaphoreType.DMA,
        pltpu.SemaphoreType.DMA,
    ],
    compiler_params=pltpu.CompilerParams(needs_layout_passes=False),
)


@jax.jit
def kernel(x):
    x32 = x.astype(jnp.int32).reshape(-1)
    out_t = _sc_call(x32)
    return out_t.transpose(0, 2, 1)


# speedup vs baseline: 3.7937x; 1.1659x over previous
import jax
import jax.numpy as jnp
from jax import lax
from jax.experimental import pallas as pl
from jax.experimental.pallas import tpu as pltpu
from jax.experimental.pallas import tpu_sc as plsc

L = 20
B = 1024
T = 1000
N = L * B
NC = 2
NS = 16
NW = NC * NS
TCR = 40
NT = T // TCR
NBLK = L * NT
NG = B // 16


def _body(x_hbm, out_hbm, x_v, b0_v, b1_v, s0, s1):
    bufs = (b0_v, b1_v)
    sems = (s0, s1)
    wid = lax.axis_index("s") * NC + lax.axis_index("c")

    xcopy = pltpu.async_copy(x_hbm, x_v, s0)
    zeros16 = jnp.zeros((16,), jnp.float32)

    def zbody(r, c):
        for k in range(2):
            for g in range(NG):
                bufs[k][r, pl.ds(g * 16, 16)] = zeros16
        return c

    lax.fori_loop(0, TCR, zbody, 0)
    xcopy.wait()

    lane = lax.iota(jnp.int32, 16)
    ones16 = jnp.ones((16,), jnp.float32)

    start = (wid * NBLK) // NW
    n = ((wid + 1) * NBLK) // NW - start

    def loc(i):
        m = start + i
        l = m // NT
        t0 = (m - l * NT) * TCR
        return l, t0

    def fill(buf, i):
        l, t0 = loc(i)

        def gbody(g, c):
            xv = x_v[l, pl.ds(g * 16, 16)]
            rel = xv - t0
            inb = (rel >= 0) & (rel < TCR)
            plsc.store_scatter(buf, [rel, g * 16 + lane], ones16, mask=inb)
            return c

        lax.fori_loop(0, NG, gbody, 0)

        @pl.when(t0 == T - TCR)
        def _():
            def cbody(g, c):
                xv = x_v[l, pl.ds(g * 16, 16)]
                valid = xv >= 0
                cnt = jnp.zeros((16,), jnp.int32)
                for j in range(L):
                    xj = x_v[j, pl.ds(g * 16, 16)]
                    hit = (xj == xv) & (j <= l)
                    cnt = cnt + hit.astype(jnp.int32)
                yv = (valid & ((cnt & 1) == 0)).astype(jnp.float32)
                cs = pl.ds(g * 16, 16)
                buf[TCR - 1, cs] = buf[TCR - 1, cs] + yv
                return c

            lax.fori_loop(0, NG, cbody, 0)

    def undo(buf, i):
        l, t0 = loc(i)

        def gbody(g, c):
            xv = x_v[l, pl.ds(g * 16, 16)]
            rel = xv - t0
            inb = (rel >= 0) & (rel < TCR)
            plsc.store_scatter(buf, [rel, g * 16 + lane], zeros16, mask=inb)
            return c

        lax.fori_loop(0, NG, gbody, 0)

        @pl.when(t0 == T - TCR)
        def _():
            def zctx(g, c):
                buf[TCR - 1, pl.ds(g * 16, 16)] = zeros16
                return c

            lax.fori_loop(0, NG, zctx, 0)

    def start_dma(buf, sem, i):
        l, t0 = loc(i)
        pltpu.async_copy(buf, out_hbm.at[l, pl.ds(t0, TCR)], sem)

    def wait_dma(buf, sem, i):
        l, t0 = loc(i)
        pltpu.make_async_copy(buf, out_hbm.at[l, pl.ds(t0, TCR)], sem).wait()

    for k in range(2):
        fill(bufs[k], k)
        start_dma(bufs[k], sems[k], k)

    def round_body(r, c):
        for k in range(2):
            i = 2 + 2 * (r - 1) + k
            wait_dma(bufs[k], sems[k], i - 2)
            undo(bufs[k], i - 2)
            fill(bufs[k], i)
            start_dma(bufs[k], sems[k], i)
        return c

    lax.fori_loop(1, (n - 2) // 2 + 1, round_body, 0)

    @pl.when((n & 1) == 1)
    def _():
        wait_dma(bufs[0], sems[0], n - 3)
        undo(bufs[0], n - 3)
        fill(bufs[0], n - 1)
        start_dma(bufs[0], sems[0], n - 1)

    wait_dma(bufs[0], sems[0], n - 1 - ((n - 1) & 1))
    wait_dma(bufs[1], sems[1], n - 1 - (n & 1))


_mesh = plsc.VectorSubcoreMesh(
    core_axis_name="c", subcore_axis_name="s", num_cores=NC, num_subcores=NS
)

_sc_call = pl.kernel(
    _body,
    out_type=jax.ShapeDtypeStruct((L, T, B), jnp.float32),
    mesh=_mesh,
    scratch_types=[
        pltpu.VMEM((L, B), jnp.int32),
        pltpu.VMEM((TCR, B), jnp.float32),
        pltpu.VMEM((TCR, B), jnp.float32),
        pltpu.SemaphoreType.DMA,
        pltpu.SemaphoreType.DMA,
    ],
    compiler_params=pltpu.CompilerParams(needs_layout_passes=False),
)


@jax.jit
def kernel(x):
    x32 = x.astype(jnp.int32)
    out_t = _sc_call(x32)
    return out_t.transpose(0, 2, 1)


# speedup vs baseline: 3.7947x; 1.0003x over previous
import jax
import jax.numpy as jnp
from jax import lax
from jax.experimental import pallas as pl
from jax.experimental.pallas import tpu as pltpu
from jax.experimental.pallas import tpu_sc as plsc

L = 20
B = 1024
T = 1000
N = L * B
NC = 2
NS = 16
NW = NC * NS
TCR = 40
NT = T // TCR
NBLK = L * NT
NG = B // 16


def _body(x_hbm, out_hbm, x_v, b0_v, b1_v, s0, s1):
    bufs = (b0_v, b1_v)
    sems = (s0, s1)
    wid = lax.axis_index("s") * NC + lax.axis_index("c")

    xcopy = pltpu.async_copy(x_hbm, x_v, s0)
    zeros16 = jnp.zeros((16,), jnp.float32)

    def zbody(r, c):
        for k in range(2):
            for g in range(NG):
                bufs[k][r, pl.ds(g * 16, 16)] = zeros16
        return c

    lax.fori_loop(0, TCR, zbody, 0)
    xcopy.wait()

    lane = lax.iota(jnp.int32, 16)
    ones16 = jnp.ones((16,), jnp.float32)

    start = (wid * NBLK) // NW
    n = ((wid + 1) * NBLK) // NW - start

    def loc(i):
        m = start + i
        l = m // NT
        t0 = (m - l * NT) * TCR
        return l, t0

    def fill(buf, i):
        l, t0 = loc(i)

        def gbody(g, c):
            xv = x_v[l, pl.ds(g * 16, 16)]
            rel = xv - t0
            inb = (rel >= 0) & (rel < TCR)
            plsc.store_scatter(buf, [rel, g * 16 + lane], ones16, mask=inb)
            return c

        lax.fori_loop(0, NG, gbody, 0)

        @pl.when(t0 == T - TCR)
        def _():
            def cbody(g, c):
                xv = x_v[l, pl.ds(g * 16, 16)]
                valid = xv >= 0
                cnt = jnp.zeros((16,), jnp.int32)
                for j in range(L):
                    xj = x_v[j, pl.ds(g * 16, 16)]
                    hit = (xj == xv) & (j <= l)
                    cnt = cnt + hit.astype(jnp.int32)
                yv = (valid & ((cnt & 1) == 0)).astype(jnp.float32)
                cs = pl.ds(g * 16, 16)
                buf[TCR - 1, cs] = buf[TCR - 1, cs] + yv
                return c

            lax.fori_loop(0, NG, cbody, 0)

    def undo(buf, i):
        l, t0 = loc(i)

        def gbody(g, c):
            xv = x_v[l, pl.ds(g * 16, 16)]
            rel = xv - t0
            inb = (rel >= 0) & (rel < TCR)
            plsc.store_scatter(buf, [rel, g * 16 + lane], zeros16, mask=inb)
            return c

        lax.fori_loop(0, NG, gbody, 0)

        @pl.when(t0 == T - TCR)
        def _():
            def zctx(g, c):
                buf[TCR - 1, pl.ds(g * 16, 16)] = zeros16
                return c

            lax.fori_loop(0, NG, zctx, 0)

    def start_dma(buf, sem, i):
        l, t0 = loc(i)
        pltpu.async_copy(buf, out_hbm.at[l, pl.ds(t0, TCR)], sem)

    def wait_dma(buf, sem, i):
        l, t0 = loc(i)
        pltpu.make_async_copy(buf, out_hbm.at[l, pl.ds(t0, TCR)], sem).wait()

    for k in range(2):
        fill(bufs[k], k)
        start_dma(bufs[k], sems[k], k)

    def round_body(r, c):
        for k in range(2):
            i = 2 + 2 * (r - 1) + k
            wait_dma(bufs[k], sems[k], i - 2)
            undo(bufs[k], i - 2)
            fill(bufs[k], i)
            start_dma(bufs[k], sems[k], i)
        return c

    lax.fori_loop(1, (n - 2) // 2 + 1, round_body, 0)

    @pl.when((n & 1) == 1)
    def _():
        wait_dma(bufs[0], sems[0], n - 3)
        undo(bufs[0], n - 3)
        fill(bufs[0], n - 1)
        start_dma(bufs[0], sems[0], n - 1)

    wait_dma(bufs[0], sems[0], n - 1 - ((n - 1) & 1))
    wait_dma(bufs[1], sems[1], n - 1 - (n & 1))


_mesh = plsc.VectorSubcoreMesh(
    core_axis_name="c", subcore_axis_name="s", num_cores=NC, num_subcores=NS
)

_sc_call = pl.kernel(
    _body,
    out_type=jax.ShapeDtypeStruct((L, T, B), jnp.float32),
    mesh=_mesh,
    scratch_types=[
        pltpu.VMEM((L, B), jnp.int32),
        pltpu.VMEM((TCR, B), jnp.float32),
        pltpu.VMEM((TCR, B), jnp.float32),
        pltpu.SemaphoreType.DMA,
        pltpu.SemaphoreType.DMA,
    ],
    compiler_params=pltpu.CompilerParams(
        needs_layout_passes=False,
        disable_bounds_checks=True,
        disable_semaphore_checks=True,
    ),
)


@jax.jit
def kernel(x):
    x32 = x.astype(jnp.int32)
    out_t = _sc_call(x32)
    return out_t.transpose(0, 2, 1)
